# Initial kernel scaffold; baseline (speedup 1.0000x reference)
import jax
import jax.numpy as jnp
from jax.experimental import pallas as pl


def kernel(hidden_states, edge_index):
    raise NotImplementedError("write your pallas kernel here")


# speedup vs baseline: 1.2275x; 1.2275x over previous
import functools

import jax
import jax.numpy as jnp
from jax import lax
from jax.experimental import pallas as pl
from jax.experimental.pallas import tpu as pltpu
from jax.experimental.pallas import tpu_sc as plsc

NC = 2
NS = 16
NW = NC * NS
L = 16

E_TOTAL = 320000
D = 128
E_PER_W = E_TOTAL // NW
G = 80
STEPS = E_PER_W // G
GROUPS = G // L


def _sc_body(hs_hbm, src_hbm, dst_hbm, out_hbm,
             idx_s, idx_d, rows_s, rows_d, out_v, sem_s, sem_d):
    wid = lax.axis_index("s") * NC + lax.axis_index("c")

    pltpu.sync_copy(src_hbm.at[wid], idx_s)
    pltpu.sync_copy(dst_hbm.at[wid], idx_d)

    def step_body(step, carry):
        pltpu.async_copy(hs_hbm.at[idx_s.at[step]], rows_s, sem_s).wait()
        pltpu.async_copy(hs_hbm.at[idx_d.at[step]], rows_d, sem_d).wait()

        for g in range(GROUPS):
            e16 = g * L + lax.broadcasted_iota(jnp.int32, (L,), 0)
            z = jnp.zeros((L,), jnp.float32)

            def fbody(fo, accs):
                a0, a1, a2, a3 = accs
                f0 = fo * 4
                prods = []
                for k in range(4):
                    fk = jnp.full((L,), f0 + k, jnp.int32)
                    s = plsc.load_gather(rows_s, [e16, fk])
                    d = plsc.load_gather(rows_d, [e16, fk])
                    prods.append(s * d)
                return (a0 + prods[0], a1 + prods[1],
                        a2 + prods[2], a3 + prods[3])

            a0, a1, a2, a3 = lax.fori_loop(0, D // 4, fbody, (z, z, z, z))
            v = (a0 + a1) + (a2 + a3)
            out_v[step, pl.ds(g * L, L)] = 1.0 / (1.0 + jnp.exp(-v))
        return carry

    lax.fori_loop(0, STEPS, step_body, 0)
    pltpu.sync_copy(out_v, out_hbm.at[wid])


@jax.jit
def _decode(hidden_states, src_idx, dst_idx):
    mesh = plsc.VectorSubcoreMesh(core_axis_name="c", subcore_axis_name="s")
    f = pl.kernel(
        _sc_body,
        mesh=mesh,
        out_type=jax.ShapeDtypeStruct((NW, STEPS, G), jnp.float32),
        scratch_types=[
            pltpu.VMEM((STEPS, G), jnp.int32),
            pltpu.VMEM((STEPS, G), jnp.int32),
            pltpu.VMEM((G, D), jnp.float32),
            pltpu.VMEM((G, D), jnp.float32),
            pltpu.VMEM((STEPS, G), jnp.float32),
            pltpu.SemaphoreType.DMA,
            pltpu.SemaphoreType.DMA,
        ],
        compiler_params=pltpu.CompilerParams(needs_layout_passes=False),
    )
    return f(hidden_states, src_idx, dst_idx)


def kernel(hidden_states, edge_index):
    ei = edge_index.astype(jnp.int32)
    src = ei[0].reshape(NW, STEPS, G)
    dst = ei[1].reshape(NW, STEPS, G)
    out = _decode(hidden_states, src, dst)
    return out.reshape(E_TOTAL)


# speedup vs baseline: 1.5135x; 1.2330x over previous
import functools

import jax
import jax.numpy as jnp
from jax import lax
from jax.experimental import pallas as pl
from jax.experimental.pallas import tpu as pltpu
from jax.experimental.pallas import tpu_sc as plsc

NC = 2
NS = 16
NW = NC * NS
L = 16

E_TOTAL = 320000
D = 128
E_PER_W = E_TOTAL // NW
G = 80
STEPS = E_PER_W // G
GROUPS = G // L


def _sc_body(hs_hbm, src_hbm, dst_hbm, out_hbm,
             idx_s, idx_d, rows_s, rows_d, out_v,
             sem_s0, sem_s1, sem_d0, sem_d1):
    wid = lax.axis_index("s") * NC + lax.axis_index("c")
    sem_s = (sem_s0, sem_s1)
    sem_d = (sem_d0, sem_d1)

    pltpu.sync_copy(src_hbm.at[wid], idx_s)
    pltpu.sync_copy(dst_hbm.at[wid], idx_d)

    def issue_pair(step, b):
        pltpu.async_copy(hs_hbm.at[idx_s.at[step]], rows_s.at[b], sem_s[b])
        pltpu.async_copy(hs_hbm.at[idx_d.at[step]], rows_d.at[b], sem_d[b])

    def wait_pair(step, b):
        pltpu.make_async_copy(
            hs_hbm.at[idx_s.at[step]], rows_s.at[b], sem_s[b]).wait()
        pltpu.make_async_copy(
            hs_hbm.at[idx_d.at[step]], rows_d.at[b], sem_d[b]).wait()

    def compute(step, b):
        for g in range(GROUPS):
            e16 = g * L + lax.broadcasted_iota(jnp.int32, (L,), 0)
            z = jnp.zeros((L,), jnp.float32)

            def fbody(fo, accs):
                a0, a1, a2, a3 = accs
                f0 = fo * 4
                prods = []
                for k in range(4):
                    fk = jnp.full((L,), f0 + k, jnp.int32)
                    s = plsc.load_gather(rows_s.at[b], [e16, fk])
                    d = plsc.load_gather(rows_d.at[b], [e16, fk])
                    prods.append(s * d)
                return (a0 + prods[0], a1 + prods[1],
                        a2 + prods[2], a3 + prods[3])

            a0, a1, a2, a3 = lax.fori_loop(0, D // 4, fbody, (z, z, z, z))
            v = (a0 + a1) + (a2 + a3)
            out_v[step, pl.ds(g * L, L)] = 1.0 / (1.0 + jnp.exp(-v))

    issue_pair(0, 0)

    def body2(i, carry):
        s0 = 2 * i
        issue_pair(s0 + 1, 1)
        wait_pair(s0, 0)
        compute(s0, 0)
        issue_pair(s0 + 2, 0)
        wait_pair(s0 + 1, 1)
        compute(s0 + 1, 1)
        return carry

    lax.fori_loop(0, (STEPS - 1) // 2, body2, 0)
    wait_pair(STEPS - 1, 0)
    compute(STEPS - 1, 0)
    pltpu.sync_copy(out_v, out_hbm.at[wid])


@jax.jit
def _decode(hidden_states, src_idx, dst_idx):
    mesh = plsc.VectorSubcoreMesh(core_axis_name="c", subcore_axis_name="s")
    f = pl.kernel(
        _sc_body,
        mesh=mesh,
        out_type=jax.ShapeDtypeStruct((NW, STEPS, G), jnp.float32),
        scratch_types=[
            pltpu.VMEM((STEPS, G), jnp.int32),
            pltpu.VMEM((STEPS, G), jnp.int32),
            pltpu.VMEM((2, G, D), jnp.float32),
            pltpu.VMEM((2, G, D), jnp.float32),
            pltpu.VMEM((STEPS, G), jnp.float32),
            pltpu.SemaphoreType.DMA,
            pltpu.SemaphoreType.DMA,
            pltpu.SemaphoreType.DMA,
            pltpu.SemaphoreType.DMA,
        ],
        compiler_params=pltpu.CompilerParams(needs_layout_passes=False),
    )
    return f(hidden_states, src_idx, dst_idx)


def kernel(hidden_states, edge_index):
    ei = edge_index.astype(jnp.int32)
    src = ei[0].reshape(NW, STEPS, G)
    dst = ei[1].reshape(NW, STEPS, G)
    out = _decode(hidden_states, src, dst)
    return out.reshape(E_TOTAL)


# speedup vs baseline: 8.7790x; 5.8006x over previous
import functools

import jax
import jax.numpy as jnp
from jax import lax
from jax.experimental import pallas as pl
from jax.experimental.pallas import tpu as pltpu
from jax.experimental.pallas import tpu_sc as plsc

NC = 2
NS = 16
NW = NC * NS
L = 16

E_TOTAL = 320000
D = 128
E_PER_W = E_TOTAL // NW
G = 80
STEPS = E_PER_W // G
GROUPS = G // L


def _sc_body(hs_hbm, src_hbm, dst_hbm, out_hbm,
             idx_s, idx_d, rows_s, rows_d, out_v,
             sem_s0, sem_s1, sem_d0, sem_d1):
    wid = lax.axis_index("s") * NC + lax.axis_index("c")
    sem_s = (sem_s0, sem_s1)
    sem_d = (sem_d0, sem_d1)

    pltpu.sync_copy(src_hbm.at[wid], idx_s)
    pltpu.sync_copy(dst_hbm.at[wid], idx_d)

    def issue_pair(step, b):
        pltpu.async_copy(hs_hbm.at[idx_s.at[step]], rows_s.at[b], sem_s[b])
        pltpu.async_copy(hs_hbm.at[idx_d.at[step]], rows_d.at[b], sem_d[b])

    def wait_pair(step, b):
        pltpu.make_async_copy(
            hs_hbm.at[idx_s.at[step]], rows_s.at[b], sem_s[b]).wait()
        pltpu.make_async_copy(
            hs_hbm.at[idx_d.at[step]], rows_d.at[b], sem_d[b]).wait()

    def compute(step, b):
        lane = lax.broadcasted_iota(jnp.int32, (L,), 0)
        for g in range(GROUPS):
            e16 = g * L + lane
            z = jnp.zeros((L,), jnp.float32)

            def fbody(fo, carry):
                a0, a1, a2, a3, fv = carry
                prods = []
                for _ in range(4):
                    s = plsc.load_gather(rows_s.at[b], [e16, fv])
                    d = plsc.load_gather(rows_d.at[b], [e16, fv])
                    prods.append(s * d)
                    fv = (fv + 1) & (D - 1)
                return (a0 + prods[0], a1 + prods[1],
                        a2 + prods[2], a3 + prods[3], fv)

            a0, a1, a2, a3, _ = lax.fori_loop(
                0, D // 4, fbody, (z, z, z, z, lane))
            v = (a0 + a1) + (a2 + a3)
            out_v[step, pl.ds(g * L, L)] = 1.0 / (1.0 + jnp.exp(-v))

    issue_pair(0, 0)

    def body2(i, carry):
        s0 = 2 * i
        issue_pair(s0 + 1, 1)
        wait_pair(s0, 0)
        compute(s0, 0)
        issue_pair(s0 + 2, 0)
        wait_pair(s0 + 1, 1)
        compute(s0 + 1, 1)
        return carry

    lax.fori_loop(0, (STEPS - 1) // 2, body2, 0)
    wait_pair(STEPS - 1, 0)
    compute(STEPS - 1, 0)
    pltpu.sync_copy(out_v, out_hbm.at[wid])


@jax.jit
def _decode(hidden_states, src_idx, dst_idx):
    mesh = plsc.VectorSubcoreMesh(core_axis_name="c", subcore_axis_name="s")
    f = pl.kernel(
        _sc_body,
        mesh=mesh,
        out_type=jax.ShapeDtypeStruct((NW, STEPS, G), jnp.float32),
        scratch_types=[
            pltpu.VMEM((STEPS, G), jnp.int32),
            pltpu.VMEM((STEPS, G), jnp.int32),
            pltpu.VMEM((2, G, D), jnp.float32),
            pltpu.VMEM((2, G, D), jnp.float32),
            pltpu.VMEM((STEPS, G), jnp.float32),
            pltpu.SemaphoreType.DMA,
            pltpu.SemaphoreType.DMA,
            pltpu.SemaphoreType.DMA,
            pltpu.SemaphoreType.DMA,
        ],
        compiler_params=pltpu.CompilerParams(needs_layout_passes=False),
    )
    return f(hidden_states, src_idx, dst_idx)


def kernel(hidden_states, edge_index):
    ei = edge_index.astype(jnp.int32)
    src = ei[0].reshape(NW, STEPS, G)
    dst = ei[1].reshape(NW, STEPS, G)
    out = _decode(hidden_states, src, dst)
    return out.reshape(E_TOTAL)


# speedup vs baseline: 10.0419x; 1.1439x over previous
import functools

import jax
import jax.numpy as jnp
from jax import lax
from jax.experimental import pallas as pl
from jax.experimental.pallas import tpu as pltpu
from jax.experimental.pallas import tpu_sc as plsc

NC = 2
NS = 16
NW = NC * NS
L = 16

E_TOTAL = 320000
D = 128
W = D // 2
E_PER_W = E_TOTAL // NW
G = 80
STEPS = E_PER_W // G
GROUPS = G // L


def _sc_body(hs_hbm, src_hbm, dst_hbm, out_hbm,
             idx_s, idx_d, rows_s, rows_d, out_v,
             sem_s0, sem_s1, sem_d0, sem_d1):
    wid = lax.axis_index("s") * NC + lax.axis_index("c")
    sem_s = (sem_s0, sem_s1)
    sem_d = (sem_d0, sem_d1)

    pltpu.sync_copy(src_hbm.at[wid], idx_s)
    pltpu.sync_copy(dst_hbm.at[wid], idx_d)

    def issue_pair(step, b):
        pltpu.async_copy(hs_hbm.at[idx_s.at[step]], rows_s.at[b], sem_s[b])
        pltpu.async_copy(hs_hbm.at[idx_d.at[step]], rows_d.at[b], sem_d[b])

    def wait_pair(step, b):
        pltpu.make_async_copy(
            hs_hbm.at[idx_s.at[step]], rows_s.at[b], sem_s[b]).wait()
        pltpu.make_async_copy(
            hs_hbm.at[idx_d.at[step]], rows_d.at[b], sem_d[b]).wait()

    def compute(step, b):
        lane = lax.broadcasted_iota(jnp.int32, (L,), 0)
        for g in range(GROUPS):
            e16 = g * L + lane
            z = jnp.zeros((L,), jnp.float32)

            def fbody(fo, carry):
                a0, a1, a2, a3, fv = carry
                accs = [a0, a1, a2, a3]
                for k in range(4):
                    s = plsc.load_gather(rows_s.at[b], [e16, fv])
                    d = plsc.load_gather(rows_d.at[b], [e16, fv])
                    sb = plsc.bitcast(s, jnp.bfloat16)
                    db = plsc.bitcast(d, jnp.bfloat16)
                    p_lo, p_hi = plsc.unpack(sb * db,
                                             format=plsc.PackFormat.INTERLEAVED)
                    accs[(2 * k) % 4] = accs[(2 * k) % 4] + p_lo
                    accs[(2 * k + 1) % 4] = accs[(2 * k + 1) % 4] + p_hi
                    fv = (fv + 1) & (W - 1)
                return (accs[0], accs[1], accs[2], accs[3], fv)

            a0, a1, a2, a3, _ = lax.fori_loop(
                0, W // 4, fbody, (z, z, z, z, lane))
            v = (a0 + a1) + (a2 + a3)
            out_v[step, pl.ds(g * L, L)] = 1.0 / (1.0 + jnp.exp(-v))

    issue_pair(0, 0)

    def body2(i, carry):
        s0 = 2 * i
        issue_pair(s0 + 1, 1)
        wait_pair(s0, 0)
        compute(s0, 0)
        issue_pair(s0 + 2, 0)
        wait_pair(s0 + 1, 1)
        compute(s0 + 1, 1)
        return carry

    lax.fori_loop(0, (STEPS - 1) // 2, body2, 0)
    wait_pair(STEPS - 1, 0)
    compute(STEPS - 1, 0)
    pltpu.sync_copy(out_v, out_hbm.at[wid])


@jax.jit
def _decode(hidden_states, src_idx, dst_idx):
    mesh = plsc.VectorSubcoreMesh(core_axis_name="c", subcore_axis_name="s")
    f = pl.kernel(
        _sc_body,
        mesh=mesh,
        out_type=jax.ShapeDtypeStruct((NW, STEPS, G), jnp.float32),
        scratch_types=[
            pltpu.VMEM((STEPS, G), jnp.int32),
            pltpu.VMEM((STEPS, G), jnp.int32),
            pltpu.VMEM((2, G, W), jnp.int32),
            pltpu.VMEM((2, G, W), jnp.int32),
            pltpu.VMEM((STEPS, G), jnp.float32),
            pltpu.SemaphoreType.DMA,
            pltpu.SemaphoreType.DMA,
            pltpu.SemaphoreType.DMA,
            pltpu.SemaphoreType.DMA,
        ],
        compiler_params=pltpu.CompilerParams(
            needs_layout_passes=False, use_tc_tiling_on_sc=False),
    )
    return f(hidden_states, src_idx, dst_idx)


def kernel(hidden_states, edge_index):
    ei = edge_index.astype(jnp.int32)
    src = ei[0].reshape(NW, STEPS, G)
    dst = ei[1].reshape(NW, STEPS, G)
    hs_bf16 = hidden_states.astype(jnp.bfloat16)
    hs_packed = jax.lax.bitcast_convert_type(
        hs_bf16.reshape(hidden_states.shape[0], W, 2), jnp.int32)
    out = _decode(hs_packed, src, dst)
    return out.reshape(E_TOTAL)


# speedup vs baseline: 11.2262x; 1.1179x over previous
import functools

import jax
import jax.numpy as jnp
from jax import lax
from jax.experimental import pallas as pl
from jax.experimental.pallas import tpu as pltpu
from jax.experimental.pallas import tpu_sc as plsc

NC = 2
NS = 16
NW = NC * NS
L = 16

E_TOTAL = 320000
D = 128
W = D // 2
E_PER_W = E_TOTAL // NW
G = 80
STEPS = E_PER_W // G
GROUPS = G // L


def _sc_body(hs_hbm, src_hbm, dst_hbm, out_hbm,
             idx_s, idx_d, rows_s, rows_d, out_v, tbl_sp,
             sem_s0, sem_s1, sem_d0, sem_d1):
    sid = lax.axis_index("s")
    wid = sid * NC + lax.axis_index("c")
    sem_s = (sem_s0, sem_s1)
    sem_d = (sem_d0, sem_d1)

    @pl.when(sid == 0)
    def _stage_table():
        pltpu.sync_copy(hs_hbm, tbl_sp)

    pltpu.sync_copy(src_hbm.at[wid], idx_s)
    pltpu.sync_copy(dst_hbm.at[wid], idx_d)
    plsc.subcore_barrier()

    def issue_pair(step, b):
        pltpu.async_copy(tbl_sp.at[idx_s.at[step]], rows_s.at[b], sem_s[b])
        pltpu.async_copy(tbl_sp.at[idx_d.at[step]], rows_d.at[b], sem_d[b])

    def wait_pair(step, b):
        pltpu.make_async_copy(
            tbl_sp.at[idx_s.at[step]], rows_s.at[b], sem_s[b]).wait()
        pltpu.make_async_copy(
            tbl_sp.at[idx_d.at[step]], rows_d.at[b], sem_d[b]).wait()

    def compute(step, b):
        lane = lax.broadcasted_iota(jnp.int32, (L,), 0)
        for g in range(GROUPS):
            e16 = g * L + lane
            z = jnp.zeros((L,), jnp.float32)

            def fbody(fo, carry):
                a0, a1, a2, a3, fv = carry
                accs = [a0, a1, a2, a3]
                for k in range(4):
                    s = plsc.load_gather(rows_s.at[b], [e16, fv])
                    d = plsc.load_gather(rows_d.at[b], [e16, fv])
                    sb = plsc.bitcast(s, jnp.bfloat16)
                    db = plsc.bitcast(d, jnp.bfloat16)
                    p_lo, p_hi = plsc.unpack(sb * db,
                                             format=plsc.PackFormat.INTERLEAVED)
                    accs[(2 * k) % 4] = accs[(2 * k) % 4] + p_lo
                    accs[(2 * k + 1) % 4] = accs[(2 * k + 1) % 4] + p_hi
                    fv = (fv + 1) & (W - 1)
                return (accs[0], accs[1], accs[2], accs[3], fv)

            a0, a1, a2, a3, _ = lax.fori_loop(
                0, W // 4, fbody, (z, z, z, z, lane))
            v = (a0 + a1) + (a2 + a3)
            out_v[step, pl.ds(g * L, L)] = 1.0 / (1.0 + jnp.exp(-v))

    issue_pair(0, 0)

    def body2(i, carry):
        s0 = 2 * i
        issue_pair(s0 + 1, 1)
        wait_pair(s0, 0)
        compute(s0, 0)
        issue_pair(s0 + 2, 0)
        wait_pair(s0 + 1, 1)
        compute(s0 + 1, 1)
        return carry

    lax.fori_loop(0, (STEPS - 1) // 2, body2, 0)
    wait_pair(STEPS - 1, 0)
    compute(STEPS - 1, 0)
    pltpu.sync_copy(out_v, out_hbm.at[wid])


@jax.jit
def _decode(hidden_states, src_idx, dst_idx):
    mesh = plsc.VectorSubcoreMesh(core_axis_name="c", subcore_axis_name="s")
    f = pl.kernel(
        _sc_body,
        mesh=mesh,
        out_type=jax.ShapeDtypeStruct((NW, STEPS, G), jnp.float32),
        scratch_types=[
            pltpu.VMEM((STEPS, G), jnp.int32),
            pltpu.VMEM((STEPS, G), jnp.int32),
            pltpu.VMEM((2, G, W), jnp.int32),
            pltpu.VMEM((2, G, W), jnp.int32),
            pltpu.VMEM((STEPS, G), jnp.float32),
            pltpu.VMEM_SHARED((10000, W), jnp.int32),
            pltpu.SemaphoreType.DMA,
            pltpu.SemaphoreType.DMA,
            pltpu.SemaphoreType.DMA,
            pltpu.SemaphoreType.DMA,
        ],
        compiler_params=pltpu.CompilerParams(
            needs_layout_passes=False, use_tc_tiling_on_sc=False),
    )
    return f(hidden_states, src_idx, dst_idx)


def kernel(hidden_states, edge_index):
    ei = edge_index.astype(jnp.int32)
    src = ei[0].reshape(NW, STEPS, G)
    dst = ei[1].reshape(NW, STEPS, G)
    hs_bf16 = hidden_states.astype(jnp.bfloat16)
    hs_packed = jax.lax.bitcast_convert_type(
        hs_bf16.reshape(hidden_states.shape[0], W, 2), jnp.int32)
    out = _decode(hs_packed, src, dst)
    return out.reshape(E_TOTAL)


# speedup vs baseline: 11.3990x; 1.0154x over previous
import functools

import jax
import jax.numpy as jnp
from jax import lax
from jax.experimental import pallas as pl
from jax.experimental.pallas import tpu as pltpu
from jax.experimental.pallas import tpu_sc as plsc

NC = 2
NS = 16
NW = NC * NS
L = 16

E_TOTAL = 320000
D = 128
W = D // 2
E_PER_W = E_TOTAL // NW
G = 80
STEPS = E_PER_W // G
GROUPS = G // L


def _sc_body(hs_hbm, src_hbm, dst_hbm, out_hbm,
             idx_s, idx_d, rows_s, rows_d, out_v, tbl_sp,
             sem_s0, sem_s1, sem_d0, sem_d1):
    sid = lax.axis_index("s")
    wid = sid * NC + lax.axis_index("c")
    sem_s = (sem_s0, sem_s1)
    sem_d = (sem_d0, sem_d1)

    @pl.when(sid == 0)
    def _stage_table():
        pltpu.sync_copy(hs_hbm, tbl_sp)

    pltpu.sync_copy(src_hbm.at[wid], idx_s)
    pltpu.sync_copy(dst_hbm.at[wid], idx_d)
    plsc.subcore_barrier()

    def issue_pair(step, b):
        pltpu.async_copy(tbl_sp.at[idx_s.at[step]], rows_s.at[b], sem_s[b])
        pltpu.async_copy(tbl_sp.at[idx_d.at[step]], rows_d.at[b], sem_d[b])

    def wait_pair(step, b):
        pltpu.make_async_copy(
            tbl_sp.at[idx_s.at[step]], rows_s.at[b], sem_s[b]).wait()
        pltpu.make_async_copy(
            tbl_sp.at[idx_d.at[step]], rows_d.at[b], sem_d[b]).wait()

    def compute(step, b):
        lane = lax.broadcasted_iota(jnp.int32, (L,), 0)
        off = [(lane + j) & 15 for j in range(16)]
        for g in range(GROUPS):
            e16 = g * L + lane
            z = jnp.zeros((L,), jnp.float32)

            def block(fo, accs):
                a = list(accs)
                base = fo * 16
                for j in range(16):
                    col = off[j] + base
                    s = plsc.load_gather(rows_s.at[b], [e16, col])
                    d = plsc.load_gather(rows_d.at[b], [e16, col])
                    p = plsc.bitcast(
                        plsc.bitcast(s, jnp.bfloat16)
                        * plsc.bitcast(d, jnp.bfloat16), jnp.int32)
                    p_lo = plsc.bitcast(p << 16, jnp.float32)
                    p_hi = plsc.bitcast(p & jnp.int32(-65536), jnp.float32)
                    a[(2 * j) % 8] = a[(2 * j) % 8] + p_lo
                    a[(2 * j + 1) % 8] = a[(2 * j + 1) % 8] + p_hi
                return tuple(a)

            accs = lax.fori_loop(0, W // 16, block, (z,) * 8)
            v = (((accs[0] + accs[1]) + (accs[2] + accs[3]))
                 + ((accs[4] + accs[5]) + (accs[6] + accs[7])))
            out_v[step, pl.ds(g * L, L)] = 1.0 / (1.0 + jnp.exp(-v))

    issue_pair(0, 0)

    def body2(i, carry):
        s0 = 2 * i
        issue_pair(s0 + 1, 1)
        wait_pair(s0, 0)
        compute(s0, 0)
        issue_pair(s0 + 2, 0)
        wait_pair(s0 + 1, 1)
        compute(s0 + 1, 1)
        return carry

    lax.fori_loop(0, (STEPS - 1) // 2, body2, 0)
    wait_pair(STEPS - 1, 0)
    compute(STEPS - 1, 0)
    pltpu.sync_copy(out_v, out_hbm.at[wid])


@jax.jit
def _decode(hidden_states, src_idx, dst_idx):
    mesh = plsc.VectorSubcoreMesh(core_axis_name="c", subcore_axis_name="s")
    f = pl.kernel(
        _sc_body,
        mesh=mesh,
        out_type=jax.ShapeDtypeStruct((NW, STEPS, G), jnp.float32),
        scratch_types=[
            pltpu.VMEM((STEPS, G), jnp.int32),
            pltpu.VMEM((STEPS, G), jnp.int32),
            pltpu.VMEM((2, G, W), jnp.int32),
            pltpu.VMEM((2, G, W), jnp.int32),
            pltpu.VMEM((STEPS, G), jnp.float32),
            pltpu.VMEM_SHARED((10000, W), jnp.int32),
            pltpu.SemaphoreType.DMA,
            pltpu.SemaphoreType.DMA,
            pltpu.SemaphoreType.DMA,
            pltpu.SemaphoreType.DMA,
        ],
        compiler_params=pltpu.CompilerParams(
            needs_layout_passes=False, use_tc_tiling_on_sc=False),
    )
    return f(hidden_states, src_idx, dst_idx)


def kernel(hidden_states, edge_index):
    ei = edge_index.astype(jnp.int32)
    src = ei[0].reshape(NW, STEPS, G)
    dst = ei[1].reshape(NW, STEPS, G)
    hs_bf16 = hidden_states.astype(jnp.bfloat16)
    hs_packed = jax.lax.bitcast_convert_type(
        hs_bf16.reshape(hidden_states.shape[0], W, 2), jnp.int32)
    out = _decode(hs_packed, src, dst)
    return out.reshape(E_TOTAL)
